# baseline (speedup 1.0000x reference)
import functools

import jax
import jax.numpy as jnp
from jax import lax
from jax.experimental import pallas as pl
from jax.experimental.pallas import tpu as pltpu
from jax.experimental.pallas import tpu_sc as plsc

N = 10000
E = 320000
D1 = 128
D2 = 64

NC = 2
NS = 16
NW = NC * NS
BLK = 128
NBLK = 79
EW = NBLK * BLK
E_PAD = NW * EW
ACC = 10240
RPT = ACC // NS
CROWS = ACC // 128
BM = 1024


def _mesh():
    return plsc.VectorSubcoreMesh(
        core_axis_name="c", subcore_axis_name="s", num_cores=NC, num_subcores=NS
    )


def _make_agg(with_counts):
    out_type = [jax.ShapeDtypeStruct((NC, ACC, D1), jnp.float32)]
    scratch = [
        pltpu.VMEM((2, BLK), jnp.int32),
        pltpu.VMEM((BLK, D1), jnp.float32),
        pltpu.VMEM_SHARED((ACC, D1), jnp.float32),
        pltpu.SemaphoreType.DMA,
    ]
    if with_counts:
        out_type.append(jax.ShapeDtypeStruct((NC, CROWS, 128), jnp.float32))
        scratch += [
            pltpu.VMEM((CROWS, 128), jnp.float32),
            pltpu.VMEM_SHARED((CROWS, 128), jnp.float32),
            pltpu.VMEM((CROWS,), jnp.int32),
        ]

    def body(*refs):
        if with_counts:
            (tab, eidx, zacc, zcnt, iden,
             p_out, c_out, eb, rows, acc_sh, sem, cnt_v, cnt_sh, id_v) = refs
        else:
            (tab, eidx, zacc,
             p_out, eb, rows, acc_sh, sem) = refs
        c = lax.axis_index("c")
        s = lax.axis_index("s")
        r0 = s * RPT
        pltpu.sync_copy(zacc.at[pl.ds(r0, RPT)], acc_sh.at[pl.ds(r0, RPT)])
        if with_counts:
            @pl.when(s < CROWS // 8)
            def _():
                pltpu.sync_copy(zcnt.at[pl.ds(s * 8, 8)],
                                cnt_sh.at[pl.ds(s * 8, 8)])
            pltpu.sync_copy(zcnt, cnt_v)
            pltpu.sync_copy(iden, id_v)
        plsc.subcore_barrier()
        w = c * NS + s
        ones16 = jnp.full((16,), 1.0, jnp.float32)

        def step(b, carry):
            pltpu.sync_copy(eidx.at[w, b], eb)
            gather = pltpu.async_copy(tab.at[eb.at[0]], rows, sem)
            if with_counts:
                for j in range(BLK // 16):
                    iv = eb[1, pl.ds(j * 16, 16)]
                    plsc.addupdate_scatter(
                        cnt_v,
                        [jnp.right_shift(iv, 7), jnp.bitwise_and(iv, 127)],
                        ones16,
                    )
            gather.wait()
            pltpu.sync_copy(rows, acc_sh.at[eb.at[1]], add=True)
            return carry

        lax.fori_loop(0, NBLK, step, 0)
        if with_counts:
            pltpu.sync_copy(cnt_v, cnt_sh.at[id_v], add=True)
        plsc.subcore_barrier()
        pltpu.sync_copy(acc_sh.at[pl.ds(r0, RPT)], p_out.at[c, pl.ds(r0, RPT)])
        if with_counts:
            @pl.when(s < CROWS // 8)
            def _():
                pltpu.sync_copy(cnt_sh.at[pl.ds(s * 8, 8)],
                                c_out.at[c, pl.ds(s * 8, 8)])

    params = pltpu.CompilerParams(needs_layout_passes=False) if with_counts else None
    return pl.kernel(
        body,
        out_type=tuple(out_type) if with_counts else out_type[0],
        mesh=_mesh(),
        compiler_params=params,
        scratch_types=scratch,
    )


def _tc1(p0, p1, c0, c1, xp, w1lT, b1, w1rT):
    nb = ACC // BM

    def body(p0r, p1r, c0r, c1r, xr, w1lr, b1r, w1rr, h_out):
        cnt = jnp.maximum(c0r[...] + c1r[...], 1.0)
        mean = (p0r[...] + p1r[...]) / cnt
        h = (
            jnp.dot(mean, w1lr[...], precision=lax.Precision.HIGHEST)
            + b1r[...]
            + jnp.dot(xr[...], w1rr[...], precision=lax.Precision.HIGHEST)
        )
        h_out[...] = jnp.maximum(h, 0.0)

    row = lambda i: (i, 0)
    fixed = lambda i: (0, 0)
    return pl.pallas_call(
        body,
        grid=(nb,),
        in_specs=[
            pl.BlockSpec((BM, D1), row),
            pl.BlockSpec((BM, D1), row),
            pl.BlockSpec((BM, 1), row),
            pl.BlockSpec((BM, 1), row),
            pl.BlockSpec((BM, D1), row),
            pl.BlockSpec((D1, D1), fixed),
            pl.BlockSpec((1, D1), fixed),
            pl.BlockSpec((D1, D1), fixed),
        ],
        out_specs=pl.BlockSpec((BM, D1), row),
        out_shape=jax.ShapeDtypeStruct((ACC, D1), jnp.float32),
    )(p0, p1, c0, c1, xp, w1lT, b1, w1rT)


def _tc2a(h, w2rT, b2):
    nb = ACC // BM

    def body(hr, w2rr, b2r, out):
        out[...] = b2r[...] + jnp.dot(
            hr[...], w2rr[...], precision=lax.Precision.HIGHEST)

    row = lambda i: (i, 0)
    fixed = lambda i: (0, 0)
    return pl.pallas_call(
        body,
        grid=(nb,),
        in_specs=[
            pl.BlockSpec((BM, D1), row),
            pl.BlockSpec((D1, D2), fixed),
            pl.BlockSpec((1, D2), fixed),
        ],
        out_specs=pl.BlockSpec((BM, D2), row),
        out_shape=jax.ShapeDtypeStruct((ACC, D2), jnp.float32),
    )(h, w2rT, b2)


def _tc2(q0, q1, c0, c1, r2, w2lT):
    nb = ACC // BM

    def body(q0r, q1r, c0r, c1r, r2r, w2lr, out):
        cnt = jnp.maximum(c0r[...] + c1r[...], 1.0)
        mean2 = (q0r[...] + q1r[...]) / cnt
        z = (
            jnp.dot(mean2, w2lr[...], precision=lax.Precision.HIGHEST)
            + r2r[...]
        )
        m = jnp.max(z, axis=1, keepdims=True)
        e = z - m
        out[...] = e - jnp.log(jnp.sum(jnp.exp(e), axis=1, keepdims=True))

    row = lambda i: (i, 0)
    fixed = lambda i: (0, 0)
    return pl.pallas_call(
        body,
        grid=(nb,),
        in_specs=[
            pl.BlockSpec((BM, D1), row),
            pl.BlockSpec((BM, D1), row),
            pl.BlockSpec((BM, 1), row),
            pl.BlockSpec((BM, 1), row),
            pl.BlockSpec((BM, D2), row),
            pl.BlockSpec((D1, D2), fixed),
        ],
        out_specs=pl.BlockSpec((BM, D2), row),
        out_shape=jax.ShapeDtypeStruct((ACC, D2), jnp.float32),
    )(q0, q1, c0, c1, r2, w2lT)


def kernel(x, edge_index, W1l, b1l, W1r, W2l, b2l, W2r):
    src = edge_index[0].astype(jnp.int32)
    dst = edge_index[1].astype(jnp.int32)
    pad = E_PAD - E
    src_p = jnp.concatenate([src, jnp.zeros((pad,), jnp.int32)])
    dummy = N + (jnp.arange(pad, dtype=jnp.int32) % (ACC - N))
    dst_p = jnp.concatenate([dst, dummy])
    eidx = jnp.stack(
        [src_p.reshape(NW, NBLK, BLK), dst_p.reshape(NW, NBLK, BLK)], axis=2)

    zacc = jnp.zeros((ACC, D1), jnp.float32)
    zcnt = jnp.zeros((CROWS, 128), jnp.float32)
    iden = jnp.arange(CROWS, dtype=jnp.int32)
    P, C = _make_agg(True)(x, eidx, zacc, zcnt, iden)
    c0 = C[0].reshape(ACC, 1)
    c1 = C[1].reshape(ACC, 1)

    xp = jnp.concatenate([x, jnp.zeros((ACC - N, D1), jnp.float32)])
    h = _tc1(P[0], P[1], c0, c1, xp, W1l.T, b1l[None, :], W1r.T)

    r2 = _tc2a(h, W2r.T, b2l[None, :])
    Q = _make_agg(False)(h, eidx, zacc)

    out = _tc2(Q[0], Q[1], c0, c1, r2, W2l.T)
    return out[:N]


# speedup vs baseline: 1.0004x; 1.0004x over previous
import jax
import jax.numpy as jnp
from jax import lax
from jax.experimental import pallas as pl
from jax.experimental.pallas import tpu as pltpu
from jax.experimental.pallas import tpu_sc as plsc

N = 10000
E = 320000
D1 = 128
D2 = 64

NC = 2
NS = 16
NW = NC * NS
BLK = 128
NBLK = 79
E_PAD = NW * NBLK * BLK
ACC = 10240
RPT = ACC // NS
CROWS = ACC // 128
BM = 1024


def _mesh():
    return plsc.VectorSubcoreMesh(
        core_axis_name="c", subcore_axis_name="s", num_cores=NC, num_subcores=NS
    )


def _make_agg(with_counts):
    out_type = [jax.ShapeDtypeStruct((NC, ACC, D1), jnp.float32)]
    scratch = [
        pltpu.VMEM((2, BLK), jnp.int32),
        pltpu.VMEM((BLK, D1), jnp.float32),
        pltpu.VMEM_SHARED((ACC, D1), jnp.float32),
        pltpu.SemaphoreType.DMA,
    ]
    if with_counts:
        out_type.append(jax.ShapeDtypeStruct((NC, CROWS, 128), jnp.float32))
        scratch += [
            pltpu.VMEM((CROWS, 128), jnp.float32),
            pltpu.VMEM_SHARED((CROWS, 128), jnp.float32),
            pltpu.VMEM((CROWS,), jnp.int32),
        ]

    def body(*refs):
        if with_counts:
            (tab, eidx, zacc, zcnt, iden,
             p_out, c_out, eb, rows, acc_sh, sem, cnt_v, cnt_sh, id_v) = refs
        else:
            (tab, eidx, zacc,
             p_out, eb, rows, acc_sh, sem) = refs
        c = lax.axis_index("c")
        s = lax.axis_index("s")
        r0 = s * RPT
        pltpu.sync_copy(zacc.at[pl.ds(r0, RPT)], acc_sh.at[pl.ds(r0, RPT)])
        if with_counts:
            @pl.when(s < CROWS // 8)
            def _():
                pltpu.sync_copy(zcnt.at[pl.ds(s * 8, 8)],
                                cnt_sh.at[pl.ds(s * 8, 8)])
            pltpu.sync_copy(zcnt, cnt_v)
            pltpu.sync_copy(iden, id_v)
        plsc.subcore_barrier()
        w = c * NS + s
        ones16 = jnp.full((16,), 1.0, jnp.float32)

        def step(b, carry):
            pltpu.sync_copy(eidx.at[w, b], eb)
            gather = pltpu.async_copy(tab.at[eb.at[0]], rows, sem)
            if with_counts:
                for j in range(BLK // 16):
                    iv = eb[1, pl.ds(j * 16, 16)]
                    plsc.addupdate_scatter(
                        cnt_v,
                        [jnp.right_shift(iv, 7), jnp.bitwise_and(iv, 127)],
                        ones16,
                    )
            gather.wait()
            pltpu.sync_copy(rows, acc_sh.at[eb.at[1]], add=True)
            return carry

        lax.fori_loop(0, NBLK, step, 0)
        if with_counts:
            pltpu.sync_copy(cnt_v, cnt_sh.at[id_v], add=True)
        plsc.subcore_barrier()
        pltpu.sync_copy(acc_sh.at[pl.ds(r0, RPT)], p_out.at[c, pl.ds(r0, RPT)])
        if with_counts:
            @pl.when(s < CROWS // 8)
            def _():
                pltpu.sync_copy(cnt_sh.at[pl.ds(s * 8, 8)],
                                c_out.at[c, pl.ds(s * 8, 8)])

    params = pltpu.CompilerParams(needs_layout_passes=False) if with_counts else None
    return pl.kernel(
        body,
        out_type=tuple(out_type) if with_counts else out_type[0],
        mesh=_mesh(),
        compiler_params=params,
        scratch_types=scratch,
    )


def _tc1(p0, p1, c0, c1, xp, w1lT, b1, w1rT):
    nb = ACC // BM

    def body(p0r, p1r, c0r, c1r, xr, w1lr, b1r, w1rr, h_out):
        cnt = jnp.maximum(c0r[...] + c1r[...], 1.0)
        mean = (p0r[...] + p1r[...]) / cnt
        h = (
            jnp.dot(mean, w1lr[...], precision=lax.Precision.HIGHEST)
            + b1r[...]
            + jnp.dot(xr[...], w1rr[...], precision=lax.Precision.HIGHEST)
        )
        h_out[...] = jnp.maximum(h, 0.0)

    row = lambda i: (i, 0)
    fixed = lambda i: (0, 0)
    return pl.pallas_call(
        body,
        grid=(nb,),
        in_specs=[
            pl.BlockSpec((BM, D1), row),
            pl.BlockSpec((BM, D1), row),
            pl.BlockSpec((BM, 1), row),
            pl.BlockSpec((BM, 1), row),
            pl.BlockSpec((BM, D1), row),
            pl.BlockSpec((D1, D1), fixed),
            pl.BlockSpec((1, D1), fixed),
            pl.BlockSpec((D1, D1), fixed),
        ],
        out_specs=pl.BlockSpec((BM, D1), row),
        out_shape=jax.ShapeDtypeStruct((ACC, D1), jnp.float32),
    )(p0, p1, c0, c1, xp, w1lT, b1, w1rT)


def _tc2a(h, w2rT, b2):
    nb = ACC // BM

    def body(hr, w2rr, b2r, out):
        out[...] = b2r[...] + jnp.dot(
            hr[...], w2rr[...], precision=lax.Precision.HIGHEST)

    row = lambda i: (i, 0)
    fixed = lambda i: (0, 0)
    return pl.pallas_call(
        body,
        grid=(nb,),
        in_specs=[
            pl.BlockSpec((BM, D1), row),
            pl.BlockSpec((D1, D2), fixed),
            pl.BlockSpec((1, D2), fixed),
        ],
        out_specs=pl.BlockSpec((BM, D2), row),
        out_shape=jax.ShapeDtypeStruct((ACC, D2), jnp.float32),
    )(h, w2rT, b2)


def _tc2(q0, q1, c0, c1, r2, w2lT):
    nb = ACC // BM

    def body(q0r, q1r, c0r, c1r, r2r, w2lr, out):
        cnt = jnp.maximum(c0r[...] + c1r[...], 1.0)
        mean2 = (q0r[...] + q1r[...]) / cnt
        z = (
            jnp.dot(mean2, w2lr[...], precision=lax.Precision.HIGHEST)
            + r2r[...]
        )
        m = jnp.max(z, axis=1, keepdims=True)
        e = z - m
        out[...] = e - jnp.log(jnp.sum(jnp.exp(e), axis=1, keepdims=True))

    row = lambda i: (i, 0)
    fixed = lambda i: (0, 0)
    return pl.pallas_call(
        body,
        grid=(nb,),
        in_specs=[
            pl.BlockSpec((BM, D1), row),
            pl.BlockSpec((BM, D1), row),
            pl.BlockSpec((BM, 1), row),
            pl.BlockSpec((BM, 1), row),
            pl.BlockSpec((BM, D2), row),
            pl.BlockSpec((D1, D2), fixed),
        ],
        out_specs=pl.BlockSpec((BM, D2), row),
        out_shape=jax.ShapeDtypeStruct((ACC, D2), jnp.float32),
    )(q0, q1, c0, c1, r2, w2lT)


def kernel(x, edge_index, W1l, b1l, W1r, W2l, b2l, W2r):
    src = edge_index[0].astype(jnp.int32)
    dst = edge_index[1].astype(jnp.int32)
    pad = E_PAD - E
    src_p = jnp.concatenate([src, jnp.zeros((pad,), jnp.int32)])
    dummy = N + (jnp.arange(pad, dtype=jnp.int32) % (ACC - N))
    dst_p = jnp.concatenate([dst, dummy])
    eidx = jnp.stack(
        [src_p.reshape(NW, NBLK, BLK), dst_p.reshape(NW, NBLK, BLK)], axis=2)

    zacc = jnp.zeros((ACC, D1), jnp.float32)
    zcnt = jnp.zeros((CROWS, 128), jnp.float32)
    iden = jnp.arange(CROWS, dtype=jnp.int32)
    P, C = _make_agg(True)(x, eidx, zacc, zcnt, iden)
    c0 = C[0].reshape(ACC, 1)
    c1 = C[1].reshape(ACC, 1)

    xp = jnp.concatenate([x, jnp.zeros((ACC - N, D1), jnp.float32)])
    h = _tc1(P[0], P[1], c0, c1, xp, W1l.T, b1l[None, :], W1r.T)

    r2 = _tc2a(h, W2r.T, b2l[None, :])
    Q = _make_agg(False)(h, eidx, zacc)

    out = _tc2(Q[0], Q[1], c0, c1, r2, W2l.T)
    return out[:N]
